# speedup vs baseline: 1.3962x; 1.3962x over previous
import jax
import jax.numpy as jnp
from jax import lax
from jax.experimental import pallas as pl
from jax.experimental.pallas import tpu as pltpu

_POOL = 2


def _pool_wta_body(x_ref, o_ref):
    xv = x_ref[0]
    H, W, T = xv.shape
    OH, OW = H // _POOL, W // _POOL
    t_idx = lax.broadcasted_iota(jnp.int32, (H, W, T), 2)
    masked = jnp.where(xv > 0.0, t_idx, T)
    first = jnp.min(masked, axis=-1)
    fh = jnp.min(first.reshape(OH, _POOL, W), axis=1)
    fp = jnp.min(fh.reshape(OH, OW, _POOL), axis=-1)
    t_out = lax.broadcasted_iota(jnp.int32, (OH, OW, T), 2)
    o_ref[0] = (t_out == fp[:, :, None]).astype(jnp.float32)


def kernel(x):
    B, C, H, W, T = x.shape
    OH, OW = H // _POOL, W // _POOL
    xr = x.reshape(B * C, H, W, T)
    out = pl.pallas_call(
        _pool_wta_body,
        out_shape=jax.ShapeDtypeStruct((B * C, OH, OW, T), jnp.float32),
        grid=(B * C,),
        in_specs=[pl.BlockSpec((1, H, W, T), lambda i: (i, 0, 0, 0))],
        out_specs=pl.BlockSpec((1, OH, OW, T), lambda i: (i, 0, 0, 0)),
        compiler_params=pltpu.CompilerParams(
            dimension_semantics=("parallel",),
        ),
        name="snn_maxpool2d_wta",
    )(xr)
    return out.reshape(B, C, OH, OW, T)


# speedup vs baseline: 3.3725x; 2.4156x over previous
import jax
import jax.numpy as jnp
from jax import lax
from jax.experimental import pallas as pl
from jax.experimental.pallas import tpu as pltpu

_POOL = 2


def _pool_wta_body(x_ref, o_ref):
    xv = x_ref[0]
    H, W, T = xv.shape
    OH, OW = H // _POOL, W // _POOL
    sh = jnp.sum(xv.reshape(OH, _POOL, W, T), axis=1)
    w_idx = lax.broadcasted_iota(jnp.int32, (OW, W), 1)
    v_idx = lax.broadcasted_iota(jnp.int32, (OW, W), 0)
    pmat = (w_idx // _POOL == v_idx).astype(jnp.float32)
    pb = jnp.broadcast_to(pmat, (OH, OW, W))
    c = lax.dot_general(pb, sh, (((2,), (1,)), ((0,), (0,))),
                        preferred_element_type=jnp.float32)
    c2 = c.reshape(OH * OW, T)
    s_idx = lax.broadcasted_iota(jnp.int32, (T, T), 0)
    t_idx = lax.broadcasted_iota(jnp.int32, (T, T), 1)
    tri = (s_idx <= t_idx).astype(jnp.float32)
    prefix = jnp.dot(c2, tri, preferred_element_type=jnp.float32)
    out = jnp.where((c2 > 0.0) & (prefix == c2), 1.0, 0.0)
    o_ref[0] = out.reshape(OH, OW, T)


def kernel(x):
    B, C, H, W, T = x.shape
    OH, OW = H // _POOL, W // _POOL
    xr = x.reshape(B * C, H, W, T)
    out = pl.pallas_call(
        _pool_wta_body,
        out_shape=jax.ShapeDtypeStruct((B * C, OH, OW, T), jnp.float32),
        grid=(B * C,),
        in_specs=[pl.BlockSpec((1, H, W, T), lambda i: (i, 0, 0, 0))],
        out_specs=pl.BlockSpec((1, OH, OW, T), lambda i: (i, 0, 0, 0)),
        compiler_params=pltpu.CompilerParams(
            dimension_semantics=("parallel",),
        ),
        name="snn_maxpool2d_wta",
    )(xr)
    return out.reshape(B, C, OH, OW, T)


# speedup vs baseline: 5.0183x; 1.4880x over previous
import jax
import jax.numpy as jnp
from jax import lax
from jax.experimental import pallas as pl
from jax.experimental.pallas import tpu as pltpu

_POOL = 2
_BLK = 4


def _pool_wta_body(x_ref, o_ref):
    xv = x_ref[...]
    N, H, W, T = xv.shape
    OH, OW = H // _POOL, W // _POOL
    sh = jnp.sum(xv.reshape(N * OH, _POOL, W, T), axis=1)
    w_idx = lax.broadcasted_iota(jnp.int32, (OW, W), 1)
    v_idx = lax.broadcasted_iota(jnp.int32, (OW, W), 0)
    pmat = (w_idx // _POOL == v_idx).astype(jnp.float32)
    pb = jnp.broadcast_to(pmat, (N * OH, OW, W))
    c = lax.dot_general(pb, sh, (((2,), (1,)), ((0,), (0,))),
                        preferred_element_type=jnp.float32)
    c2 = c.reshape(N * OH * OW, T)
    s_idx = lax.broadcasted_iota(jnp.int32, (T, T), 0)
    t_idx = lax.broadcasted_iota(jnp.int32, (T, T), 1)
    tri = (s_idx <= t_idx).astype(jnp.float32)
    prefix = jnp.dot(c2, tri, preferred_element_type=jnp.float32)
    out = jnp.where((c2 > 0.0) & (prefix == c2), 1.0, 0.0)
    o_ref[...] = out.reshape(N, OH, OW, T)


def kernel(x):
    B, C, H, W, T = x.shape
    OH, OW = H // _POOL, W // _POOL
    BC = B * C
    xr = x.reshape(BC, H, W, T)
    out = pl.pallas_call(
        _pool_wta_body,
        out_shape=jax.ShapeDtypeStruct((BC, OH, OW, T), jnp.float32),
        grid=(BC // _BLK,),
        in_specs=[pl.BlockSpec((_BLK, H, W, T), lambda i: (i, 0, 0, 0))],
        out_specs=pl.BlockSpec((_BLK, OH, OW, T), lambda i: (i, 0, 0, 0)),
        compiler_params=pltpu.CompilerParams(
            dimension_semantics=("parallel",),
        ),
        name="snn_maxpool2d_wta",
    )(xr)
    return out.reshape(B, C, OH, OW, T)
